# baseline (speedup 1.0000x reference)
import functools

import jax
import jax.numpy as jnp
import numpy as np
from jax import lax
from jax.experimental import pallas as pl
from jax.experimental.pallas import tpu as pltpu
from jax.experimental.pallas import tpu_sc as plsc

_K = 128
_NUM_PAIR = 512
_START = 0.0
_STOP = 9.0

_MEAN = np.linspace(_START, _STOP, _K).astype(np.float32)
_STD = np.float32(_MEAN[1] - _MEAN[0])
_INV_STD = np.float32(1.0) / _STD
_A = np.float32((2.0 * 3.14159) ** 0.5)
_LN_C = np.float32(np.log(1.0 / (_A * _STD)))

_NC = 2
_NS = 16
_NW = _NC * _NS
_L = 16


def _sc_affine_body(pairs_hbm, x_hbm, mul_hbm, bias_hbm, y_hbm,
                    pairs_v, x_v, y_v, mul_v, bias_v):
    e_per_w = y_v.shape[0]
    wid = lax.axis_index("s") * _NC + lax.axis_index("c")
    base = wid * e_per_w
    pltpu.sync_copy(pairs_hbm.at[pl.ds(base * 2, e_per_w * 2)], pairs_v)
    pltpu.sync_copy(x_hbm.at[pl.ds(base, e_per_w)], x_v)
    pltpu.sync_copy(mul_hbm, mul_v)
    pltpu.sync_copy(bias_hbm, bias_v)

    lane2 = 2 * lax.iota(jnp.int32, _L)

    def body(i, _):
        pos0 = i * (2 * _L) + lane2
        i0 = plsc.load_gather(pairs_v, [pos0])
        i1 = plsc.load_gather(pairs_v, [pos0 + 1])
        m = jnp.abs(plsc.load_gather(mul_v, [i0])) + jnp.abs(
            plsc.load_gather(mul_v, [i1]))
        b = plsc.load_gather(bias_v, [i0]) + plsc.load_gather(bias_v, [i1])
        xv = x_v[pl.ds(i * _L, _L)]
        y_v[pl.ds(i * _L, _L)] = m * xv + b
        return 0

    lax.fori_loop(0, e_per_w // _L, body, 0)
    pltpu.sync_copy(y_v, y_hbm.at[pl.ds(base, e_per_w)])


def _sc_affine(pairs, x_flat, mul_t, bias_t):
    e = x_flat.shape[0]
    e_per_w = e // _NW
    mesh = plsc.VectorSubcoreMesh(core_axis_name="c", subcore_axis_name="s",
                                  num_cores=_NC, num_subcores=_NS)
    fn = pl.kernel(
        _sc_affine_body,
        out_type=jax.ShapeDtypeStruct((e,), jnp.float32),
        mesh=mesh,
        compiler_params=pltpu.CompilerParams(needs_layout_passes=False),
        scratch_types=[
            pltpu.VMEM((e_per_w * 2,), jnp.int32),
            pltpu.VMEM((e_per_w,), jnp.float32),
            pltpu.VMEM((e_per_w,), jnp.float32),
            pltpu.VMEM((_NUM_PAIR,), jnp.float32),
            pltpu.VMEM((_NUM_PAIR,), jnp.float32),
        ],
    )
    return fn(pairs, x_flat, mul_t, bias_t)


_NBUF = 4


def _tc_expand_body(nsteps, y_ref, o_hbm, o_buf, sems):
    i = pl.program_id(0)
    slot = lax.rem(i, _NBUF)
    r = o_buf.shape[1]

    @pl.when(i >= _NBUF)
    def _wait_prev():
        pltpu.make_async_copy(
            o_buf.at[slot],
            o_hbm.at[pl.ds((i - _NBUF) * r, r)],
            sems.at[slot]).wait()

    y = y_ref[...]
    shape = (r, y.shape[1], _K)
    kf = lax.broadcasted_iota(jnp.int32, shape, 2).astype(jnp.float32)
    t = y[:, :, None] * _INV_STD - kf
    o_buf[slot] = jnp.exp(t * t * np.float32(-0.5) + _LN_C)
    pltpu.make_async_copy(
        o_buf.at[slot], o_hbm.at[pl.ds(i * r, r)], sems.at[slot]).start()

    @pl.when(i == nsteps - 1)
    def _drain():
        for s in range(_NBUF):
            step = nsteps - _NBUF + s
            pltpu.make_async_copy(
                o_buf.at[step % _NBUF],
                o_hbm.at[pl.ds(step * r, r)],
                sems.at[step % _NBUF]).wait()


def _tc_expand(y2d, rows_per_block):
    rows, cols = y2d.shape
    nsteps = rows // rows_per_block
    return pl.pallas_call(
        functools.partial(_tc_expand_body, nsteps),
        grid=(nsteps,),
        in_specs=[pl.BlockSpec((rows_per_block, cols), lambda i: (i, 0))],
        out_specs=pl.BlockSpec(memory_space=pl.ANY),
        out_shape=jax.ShapeDtypeStruct((rows, cols, _K), jnp.float32),
        scratch_shapes=[
            pltpu.VMEM((_NBUF, rows_per_block, cols, _K), jnp.float32),
            pltpu.SemaphoreType.DMA((_NBUF,)),
        ],
        compiler_params=pltpu.CompilerParams(
            dimension_semantics=("arbitrary",)),
    )(y2d)


def _sc_full_body(pairs_hbm, x_hbm, mul_hbm, bias_hbm, out_hbm,
                  pairs_v, x_v, y_v, mul_v, bias_v, obuf, sems):
    e_per_w = y_v.shape[0]
    ce = obuf.shape[1] // _K
    nch = e_per_w // ce
    wid = lax.axis_index("s") * _NC + lax.axis_index("c")
    base = wid * e_per_w
    pltpu.sync_copy(pairs_hbm.at[pl.ds(base * 2, e_per_w * 2)], pairs_v)
    pltpu.sync_copy(x_hbm.at[pl.ds(base, e_per_w)], x_v)
    pltpu.sync_copy(mul_hbm, mul_v)
    pltpu.sync_copy(bias_hbm, bias_v)

    lane = lax.iota(jnp.int32, _L)
    lane2 = 2 * lane
    kf = [(lane + 16 * j).astype(jnp.float32) for j in range(_K // _L)]

    def affine(i, _):
        pos0 = i * (2 * _L) + lane2
        i0 = plsc.load_gather(pairs_v, [pos0])
        i1 = plsc.load_gather(pairs_v, [pos0 + 1])
        m = jnp.abs(plsc.load_gather(mul_v, [i0])) + jnp.abs(
            plsc.load_gather(mul_v, [i1]))
        b = plsc.load_gather(bias_v, [i0]) + plsc.load_gather(bias_v, [i1])
        xv = x_v[pl.ds(i * _L, _L)] * _INV_STD
        y_v[pl.ds(i * _L, _L)] = m * xv + b * _INV_STD
        return 0

    lax.fori_loop(0, e_per_w // _L, affine, 0)

    def out_copy(ch, slot):
        return pltpu.make_async_copy(
            obuf.at[slot],
            out_hbm.at[pl.ds((base + ch * ce) * _K, ce * _K)],
            sems.at[slot])

    for ch in range(nch):
        slot = ch % 2

        if ch >= 2:
            out_copy(ch - 2, slot).wait()

        def expand(el, _):
            eidx = jnp.full((_L,), ch * ce + el, jnp.int32)
            yb = plsc.load_gather(y_v, [eidx])
            for j in range(_K // _L):
                t = yb - kf[j]
                obuf[slot, pl.ds(el * _K + j * _L, _L)] = jnp.exp(
                    t * t * np.float32(-0.5) + _LN_C)
            return 0

        lax.fori_loop(0, ce, expand, 0)
        out_copy(ch, slot).start()

    for ch in (nch - 2, nch - 1):
        out_copy(ch, ch % 2).wait()


def _sc_full(pairs, x_flat, mul_t, bias_t, ce=256):
    e = x_flat.shape[0]
    e_per_w = e // _NW
    mesh = plsc.VectorSubcoreMesh(core_axis_name="c", subcore_axis_name="s",
                                  num_cores=_NC, num_subcores=_NS)
    fn = pl.kernel(
        _sc_full_body,
        out_type=jax.ShapeDtypeStruct((e * _K,), jnp.float32),
        mesh=mesh,
        compiler_params=pltpu.CompilerParams(needs_layout_passes=False),
        scratch_types=[
            pltpu.VMEM((e_per_w * 2,), jnp.int32),
            pltpu.VMEM((e_per_w,), jnp.float32),
            pltpu.VMEM((e_per_w,), jnp.float32),
            pltpu.VMEM((_NUM_PAIR,), jnp.float32),
            pltpu.VMEM((_NUM_PAIR,), jnp.float32),
            pltpu.VMEM((2, ce * _K), jnp.float32),
            pltpu.SemaphoreType.DMA((2,)),
        ],
    )
    return fn(pairs, x_flat, mul_t, bias_t)


def kernel(x, atom_pair, mul_weight, bias_weight):
    b, n = x.shape[0], x.shape[1]
    e = b * n * n
    pairs = atom_pair.reshape(e * 2)
    x_flat = x.reshape(e)
    mul_t = mul_weight.reshape(_NUM_PAIR)
    bias_t = bias_weight.reshape(_NUM_PAIR)
    out = _sc_full(pairs, x_flat, mul_t, bias_t)
    return out.reshape(b, n, n, _K)


# speedup vs baseline: 1.7553x; 1.7553x over previous
import functools

import jax
import jax.numpy as jnp
import numpy as np
from jax import lax
from jax.experimental import pallas as pl
from jax.experimental.pallas import tpu as pltpu
from jax.experimental.pallas import tpu_sc as plsc

_K = 128
_NUM_PAIR = 512
_START = 0.0
_STOP = 9.0

_MEAN = np.linspace(_START, _STOP, _K).astype(np.float32)
_STD = np.float32(_MEAN[1] - _MEAN[0])
_INV_STD = np.float32(1.0) / _STD
_A = np.float32((2.0 * 3.14159) ** 0.5)
_LN_C = np.float32(np.log(1.0 / (_A * _STD)))

_NC = 2
_NS = 16
_NW = _NC * _NS
_L = 16


def _sc_affine_body(pairs_hbm, x_hbm, mul_hbm, bias_hbm, y_hbm,
                    pairs_v, x_v, y_v, mul_v, bias_v):
    e_per_w = y_v.shape[0]
    wid = lax.axis_index("s") * _NC + lax.axis_index("c")
    base = wid * e_per_w
    pltpu.sync_copy(pairs_hbm.at[pl.ds(base * 2, e_per_w * 2)], pairs_v)
    pltpu.sync_copy(x_hbm.at[pl.ds(base, e_per_w)], x_v)
    pltpu.sync_copy(mul_hbm, mul_v)
    pltpu.sync_copy(bias_hbm, bias_v)

    lane2 = 2 * lax.iota(jnp.int32, _L)

    def body(i, _):
        pos0 = i * (2 * _L) + lane2
        i0 = plsc.load_gather(pairs_v, [pos0])
        i1 = plsc.load_gather(pairs_v, [pos0 + 1])
        m = jnp.abs(plsc.load_gather(mul_v, [i0])) + jnp.abs(
            plsc.load_gather(mul_v, [i1]))
        b = plsc.load_gather(bias_v, [i0]) + plsc.load_gather(bias_v, [i1])
        xv = x_v[pl.ds(i * _L, _L)]
        y_v[pl.ds(i * _L, _L)] = m * xv + b
        return 0

    lax.fori_loop(0, e_per_w // _L, body, 0)
    pltpu.sync_copy(y_v, y_hbm.at[pl.ds(base, e_per_w)])


def _sc_affine(pairs, x_flat, mul_t, bias_t):
    e = x_flat.shape[0]
    e_per_w = e // _NW
    mesh = plsc.VectorSubcoreMesh(core_axis_name="c", subcore_axis_name="s",
                                  num_cores=_NC, num_subcores=_NS)
    fn = pl.kernel(
        _sc_affine_body,
        out_type=jax.ShapeDtypeStruct((e,), jnp.float32),
        mesh=mesh,
        compiler_params=pltpu.CompilerParams(needs_layout_passes=False),
        scratch_types=[
            pltpu.VMEM((e_per_w * 2,), jnp.int32),
            pltpu.VMEM((e_per_w,), jnp.float32),
            pltpu.VMEM((e_per_w,), jnp.float32),
            pltpu.VMEM((_NUM_PAIR,), jnp.float32),
            pltpu.VMEM((_NUM_PAIR,), jnp.float32),
        ],
    )
    return fn(pairs, x_flat, mul_t, bias_t)


_NBUF = 4


def _tc_expand_body(nsteps, y_ref, o_hbm, o_buf, sems):
    i = pl.program_id(0)
    slot = lax.rem(i, _NBUF)
    r = o_buf.shape[1]

    @pl.when(i >= _NBUF)
    def _wait_prev():
        pltpu.make_async_copy(
            o_buf.at[slot],
            o_hbm.at[pl.ds((i - _NBUF) * r, r)],
            sems.at[slot]).wait()

    y = y_ref[...]
    shape = (r, y.shape[1], _K)
    kf = lax.broadcasted_iota(jnp.int32, shape, 2).astype(jnp.float32)
    t = y[:, :, None] * _INV_STD - kf
    o_buf[slot] = jnp.exp(t * t * np.float32(-0.5) + _LN_C)
    pltpu.make_async_copy(
        o_buf.at[slot], o_hbm.at[pl.ds(i * r, r)], sems.at[slot]).start()

    @pl.when(i == nsteps - 1)
    def _drain():
        for s in range(_NBUF):
            step = nsteps - _NBUF + s
            pltpu.make_async_copy(
                o_buf.at[step % _NBUF],
                o_hbm.at[pl.ds(step * r, r)],
                sems.at[step % _NBUF]).wait()


def _tc_expand(y2d, rows_per_block):
    rows, cols = y2d.shape
    nsteps = rows // rows_per_block
    return pl.pallas_call(
        functools.partial(_tc_expand_body, nsteps),
        grid=(nsteps,),
        in_specs=[pl.BlockSpec((rows_per_block, cols), lambda i: (i, 0))],
        out_specs=pl.BlockSpec(memory_space=pl.ANY),
        out_shape=jax.ShapeDtypeStruct((rows, cols, _K), jnp.float32),
        scratch_shapes=[
            pltpu.VMEM((_NBUF, rows_per_block, cols, _K), jnp.float32),
            pltpu.SemaphoreType.DMA((_NBUF,)),
        ],
        compiler_params=pltpu.CompilerParams(
            dimension_semantics=("arbitrary",)),
    )(y2d)


def _sc_full_body(pairs_hbm, x_hbm, mul_hbm, bias_hbm, out_hbm,
                  pairs_v, x_v, y_v, mul_v, bias_v, obuf, sems):
    e_per_w = y_v.shape[0]
    ce = obuf.shape[1] // _K
    nch = e_per_w // ce
    wid = lax.axis_index("s") * _NC + lax.axis_index("c")
    base = wid * e_per_w
    pltpu.sync_copy(pairs_hbm.at[pl.ds(base * 2, e_per_w * 2)], pairs_v)
    pltpu.sync_copy(x_hbm.at[pl.ds(base, e_per_w)], x_v)
    pltpu.sync_copy(mul_hbm, mul_v)
    pltpu.sync_copy(bias_hbm, bias_v)

    lane = lax.iota(jnp.int32, _L)
    lane2 = 2 * lane
    kf = [(lane + 16 * j).astype(jnp.float32) for j in range(_K // _L)]

    def affine(i, _):
        pos0 = i * (2 * _L) + lane2
        i0 = plsc.load_gather(pairs_v, [pos0])
        i1 = plsc.load_gather(pairs_v, [pos0 + 1])
        m = jnp.abs(plsc.load_gather(mul_v, [i0])) + jnp.abs(
            plsc.load_gather(mul_v, [i1]))
        b = plsc.load_gather(bias_v, [i0]) + plsc.load_gather(bias_v, [i1])
        xv = x_v[pl.ds(i * _L, _L)] * _INV_STD
        y_v[pl.ds(i * _L, _L)] = m * xv + b * _INV_STD
        return 0

    lax.fori_loop(0, e_per_w // _L, affine, 0)

    def out_copy(ch, slot):
        return pltpu.make_async_copy(
            obuf.at[slot],
            out_hbm.at[pl.ds((base + ch * ce) * _K, ce * _K)],
            sems.at[slot])

    for ch in range(nch):
        slot = ch % 2

        if ch >= 2:
            out_copy(ch - 2, slot).wait()

        def expand(el, _):
            eidx = jnp.full((_L,), ch * ce + el, jnp.int32)
            yb = plsc.load_gather(y_v, [eidx])
            for j in range(_K // _L):
                t = yb - kf[j]
                obuf[slot, pl.ds(el * _K + j * _L, _L)] = jnp.exp(
                    t * t * np.float32(-0.5) + _LN_C)
            return 0

        lax.fori_loop(0, ce, expand, 0)
        out_copy(ch, slot).start()

    for ch in (nch - 2, nch - 1):
        out_copy(ch, ch % 2).wait()


def _sc_full(pairs, x_flat, mul_t, bias_t, ce=256):
    e = x_flat.shape[0]
    e_per_w = e // _NW
    mesh = plsc.VectorSubcoreMesh(core_axis_name="c", subcore_axis_name="s",
                                  num_cores=_NC, num_subcores=_NS)
    fn = pl.kernel(
        _sc_full_body,
        out_type=jax.ShapeDtypeStruct((e * _K,), jnp.float32),
        mesh=mesh,
        compiler_params=pltpu.CompilerParams(needs_layout_passes=False),
        scratch_types=[
            pltpu.VMEM((e_per_w * 2,), jnp.int32),
            pltpu.VMEM((e_per_w,), jnp.float32),
            pltpu.VMEM((e_per_w,), jnp.float32),
            pltpu.VMEM((_NUM_PAIR,), jnp.float32),
            pltpu.VMEM((_NUM_PAIR,), jnp.float32),
            pltpu.VMEM((2, ce * _K), jnp.float32),
            pltpu.SemaphoreType.DMA((2,)),
        ],
    )
    return fn(pairs, x_flat, mul_t, bias_t)


def kernel(x, atom_pair, mul_weight, bias_weight):
    b, n = x.shape[0], x.shape[1]
    e = b * n * n
    pairs = atom_pair.reshape(e * 2)
    x_flat = x.reshape(e)
    mul_t = mul_weight.reshape(_NUM_PAIR)
    bias_t = bias_weight.reshape(_NUM_PAIR)
    y = _sc_affine(pairs, x_flat, mul_t, bias_t)
    out = _tc_expand(y.reshape(b * n, n), rows_per_block=32)
    return out.reshape(b, n, n, _K)


# speedup vs baseline: 1.8092x; 1.0307x over previous
import functools

import jax
import jax.numpy as jnp
import numpy as np
from jax import lax
from jax.experimental import pallas as pl
from jax.experimental.pallas import tpu as pltpu
from jax.experimental.pallas import tpu_sc as plsc

_K = 128
_NUM_PAIR = 512
_START = 0.0
_STOP = 9.0

_MEAN = np.linspace(_START, _STOP, _K).astype(np.float32)
_STD = np.float32(_MEAN[1] - _MEAN[0])
_INV_STD = np.float32(1.0) / _STD
_A = np.float32((2.0 * 3.14159) ** 0.5)
_LN_C = np.float32(np.log(1.0 / (_A * _STD)))

_NC = 2
_NS = 16
_NW = _NC * _NS
_L = 16


def _sc_affine_body(pairs_hbm, x_hbm, mul_hbm, bias_hbm, y_hbm,
                    pairs_v, x_v, y_v, mul_v, bias_v, insems):
    e_per_w = y_v.shape[0]
    wid = lax.axis_index("s") * _NC + lax.axis_index("c")
    base = wid * e_per_w
    copies = [
        pltpu.async_copy(pairs_hbm.at[pl.ds(base * 2, e_per_w * 2)],
                         pairs_v, insems.at[0]),
        pltpu.async_copy(x_hbm.at[pl.ds(base, e_per_w)], x_v, insems.at[1]),
        pltpu.async_copy(mul_hbm, mul_v, insems.at[2]),
        pltpu.async_copy(bias_hbm, bias_v, insems.at[3]),
    ]
    for c in copies:
        c.wait()

    lane2 = 2 * lax.iota(jnp.int32, _L)

    @plsc.parallel_loop(0, e_per_w // _L, unroll=4)
    def body(i):
        pos0 = i * (2 * _L) + lane2
        i0 = plsc.load_gather(pairs_v, [pos0])
        i1 = plsc.load_gather(pairs_v, [pos0 + 1])
        m = jnp.abs(plsc.load_gather(mul_v, [i0])) + jnp.abs(
            plsc.load_gather(mul_v, [i1]))
        b = plsc.load_gather(bias_v, [i0]) + plsc.load_gather(bias_v, [i1])
        xv = x_v[pl.ds(i * _L, _L)]
        y_v[pl.ds(i * _L, _L)] = m * xv + b

    pltpu.sync_copy(y_v, y_hbm.at[pl.ds(base, e_per_w)])


def _sc_affine(pairs, x_flat, mul_t, bias_t):
    e = x_flat.shape[0]
    e_per_w = e // _NW
    mesh = plsc.VectorSubcoreMesh(core_axis_name="c", subcore_axis_name="s",
                                  num_cores=_NC, num_subcores=_NS)
    fn = pl.kernel(
        _sc_affine_body,
        out_type=jax.ShapeDtypeStruct((e,), jnp.float32),
        mesh=mesh,
        compiler_params=pltpu.CompilerParams(needs_layout_passes=False),
        scratch_types=[
            pltpu.VMEM((e_per_w * 2,), jnp.int32),
            pltpu.VMEM((e_per_w,), jnp.float32),
            pltpu.VMEM((e_per_w,), jnp.float32),
            pltpu.VMEM((_NUM_PAIR,), jnp.float32),
            pltpu.VMEM((_NUM_PAIR,), jnp.float32),
            pltpu.SemaphoreType.DMA((4,)),
        ],
    )
    return fn(pairs, x_flat, mul_t, bias_t)


_NBUF = 4


def _tc_expand_body(nsteps, y_ref, o_hbm, o_buf, sems):
    i = pl.program_id(0)
    slot = lax.rem(i, _NBUF)
    r = o_buf.shape[1]

    @pl.when(i >= _NBUF)
    def _wait_prev():
        pltpu.make_async_copy(
            o_buf.at[slot],
            o_hbm.at[pl.ds((i - _NBUF) * r, r)],
            sems.at[slot]).wait()

    y = y_ref[...]
    shape = (r, y.shape[1], _K)
    kf = lax.broadcasted_iota(jnp.int32, shape, 2).astype(jnp.float32)
    t = y[:, :, None] * _INV_STD - kf
    o_buf[slot] = jnp.exp(t * t * np.float32(-0.5) + _LN_C)
    pltpu.make_async_copy(
        o_buf.at[slot], o_hbm.at[pl.ds(i * r, r)], sems.at[slot]).start()

    @pl.when(i == nsteps - 1)
    def _drain():
        for s in range(_NBUF):
            step = nsteps - _NBUF + s
            pltpu.make_async_copy(
                o_buf.at[step % _NBUF],
                o_hbm.at[pl.ds(step * r, r)],
                sems.at[step % _NBUF]).wait()


def _tc_expand(y2d, rows_per_block):
    rows, cols = y2d.shape
    nsteps = rows // rows_per_block
    return pl.pallas_call(
        functools.partial(_tc_expand_body, nsteps),
        grid=(nsteps,),
        in_specs=[pl.BlockSpec((rows_per_block, cols), lambda i: (i, 0))],
        out_specs=pl.BlockSpec(memory_space=pl.ANY),
        out_shape=jax.ShapeDtypeStruct((rows, cols, _K), jnp.float32),
        scratch_shapes=[
            pltpu.VMEM((_NBUF, rows_per_block, cols, _K), jnp.float32),
            pltpu.SemaphoreType.DMA((_NBUF,)),
        ],
        compiler_params=pltpu.CompilerParams(
            dimension_semantics=("arbitrary",)),
    )(y2d)


def _sc_full_body(pairs_hbm, x_hbm, mul_hbm, bias_hbm, out_hbm,
                  pairs_v, x_v, y_v, mul_v, bias_v, obuf, sems):
    e_per_w = y_v.shape[0]
    ce = obuf.shape[1] // _K
    nch = e_per_w // ce
    wid = lax.axis_index("s") * _NC + lax.axis_index("c")
    base = wid * e_per_w
    pltpu.sync_copy(pairs_hbm.at[pl.ds(base * 2, e_per_w * 2)], pairs_v)
    pltpu.sync_copy(x_hbm.at[pl.ds(base, e_per_w)], x_v)
    pltpu.sync_copy(mul_hbm, mul_v)
    pltpu.sync_copy(bias_hbm, bias_v)

    lane = lax.iota(jnp.int32, _L)
    lane2 = 2 * lane
    kf = [(lane + 16 * j).astype(jnp.float32) for j in range(_K // _L)]

    def affine(i, _):
        pos0 = i * (2 * _L) + lane2
        i0 = plsc.load_gather(pairs_v, [pos0])
        i1 = plsc.load_gather(pairs_v, [pos0 + 1])
        m = jnp.abs(plsc.load_gather(mul_v, [i0])) + jnp.abs(
            plsc.load_gather(mul_v, [i1]))
        b = plsc.load_gather(bias_v, [i0]) + plsc.load_gather(bias_v, [i1])
        xv = x_v[pl.ds(i * _L, _L)] * _INV_STD
        y_v[pl.ds(i * _L, _L)] = m * xv + b * _INV_STD
        return 0

    lax.fori_loop(0, e_per_w // _L, affine, 0)

    def out_copy(ch, slot):
        return pltpu.make_async_copy(
            obuf.at[slot],
            out_hbm.at[pl.ds((base + ch * ce) * _K, ce * _K)],
            sems.at[slot])

    for ch in range(nch):
        slot = ch % 2

        if ch >= 2:
            out_copy(ch - 2, slot).wait()

        def expand(el, _):
            eidx = jnp.full((_L,), ch * ce + el, jnp.int32)
            yb = plsc.load_gather(y_v, [eidx])
            for j in range(_K // _L):
                t = yb - kf[j]
                obuf[slot, pl.ds(el * _K + j * _L, _L)] = jnp.exp(
                    t * t * np.float32(-0.5) + _LN_C)
            return 0

        lax.fori_loop(0, ce, expand, 0)
        out_copy(ch, slot).start()

    for ch in (nch - 2, nch - 1):
        out_copy(ch, ch % 2).wait()


def _sc_full(pairs, x_flat, mul_t, bias_t, ce=256):
    e = x_flat.shape[0]
    e_per_w = e // _NW
    mesh = plsc.VectorSubcoreMesh(core_axis_name="c", subcore_axis_name="s",
                                  num_cores=_NC, num_subcores=_NS)
    fn = pl.kernel(
        _sc_full_body,
        out_type=jax.ShapeDtypeStruct((e * _K,), jnp.float32),
        mesh=mesh,
        compiler_params=pltpu.CompilerParams(needs_layout_passes=False),
        scratch_types=[
            pltpu.VMEM((e_per_w * 2,), jnp.int32),
            pltpu.VMEM((e_per_w,), jnp.float32),
            pltpu.VMEM((e_per_w,), jnp.float32),
            pltpu.VMEM((_NUM_PAIR,), jnp.float32),
            pltpu.VMEM((_NUM_PAIR,), jnp.float32),
            pltpu.VMEM((2, ce * _K), jnp.float32),
            pltpu.SemaphoreType.DMA((2,)),
        ],
    )
    return fn(pairs, x_flat, mul_t, bias_t)


def kernel(x, atom_pair, mul_weight, bias_weight):
    b, n = x.shape[0], x.shape[1]
    e = b * n * n
    pairs = atom_pair.reshape(e * 2)
    x_flat = x.reshape(e)
    mul_t = mul_weight.reshape(_NUM_PAIR)
    bias_t = bias_weight.reshape(_NUM_PAIR)
    y = _sc_affine(pairs, x_flat, mul_t, bias_t)
    out = _tc_expand(y.reshape(b * n, n), rows_per_block=32)
    return out.reshape(b, n, n, _K)


# speedup vs baseline: 1.8094x; 1.0001x over previous
import functools

import jax
import jax.numpy as jnp
import numpy as np
from jax import lax
from jax.experimental import pallas as pl
from jax.experimental.pallas import tpu as pltpu
from jax.experimental.pallas import tpu_sc as plsc

_K = 128
_NUM_PAIR = 512
_START = 0.0
_STOP = 9.0

_MEAN = np.linspace(_START, _STOP, _K).astype(np.float32)
_STD = np.float32(_MEAN[1] - _MEAN[0])
_INV_STD = np.float32(1.0) / _STD
_A = np.float32((2.0 * 3.14159) ** 0.5)
_LN_C = np.float32(np.log(1.0 / (_A * _STD)))

_NC = 2
_NS = 16
_NW = _NC * _NS
_L = 16


def _sc_affine_body(pairs_hbm, x_hbm, mul_hbm, bias_hbm, y_hbm,
                    pairs_v, x_v, y_v, mul_v, bias_v, insems):
    e_per_w = y_v.shape[0]
    wid = lax.axis_index("s") * _NC + lax.axis_index("c")
    base = wid * e_per_w
    copies = [
        pltpu.async_copy(pairs_hbm.at[pl.ds(base * 2, e_per_w * 2)],
                         pairs_v, insems.at[0]),
        pltpu.async_copy(x_hbm.at[pl.ds(base, e_per_w)], x_v, insems.at[1]),
        pltpu.async_copy(mul_hbm, mul_v, insems.at[2]),
        pltpu.async_copy(bias_hbm, bias_v, insems.at[3]),
    ]
    for c in copies:
        c.wait()

    lane2 = 2 * lax.iota(jnp.int32, _L)

    @plsc.parallel_loop(0, e_per_w // _L, unroll=8)
    def body(i):
        pos0 = i * (2 * _L) + lane2
        i0 = plsc.load_gather(pairs_v, [pos0])
        i1 = plsc.load_gather(pairs_v, [pos0 + 1])
        m = jnp.abs(plsc.load_gather(mul_v, [i0])) + jnp.abs(
            plsc.load_gather(mul_v, [i1]))
        b = plsc.load_gather(bias_v, [i0]) + plsc.load_gather(bias_v, [i1])
        xv = x_v[pl.ds(i * _L, _L)]
        y_v[pl.ds(i * _L, _L)] = m * xv + b

    pltpu.sync_copy(y_v, y_hbm.at[pl.ds(base, e_per_w)])


def _sc_affine(pairs, x_flat, mul_t, bias_t):
    e = x_flat.shape[0]
    e_per_w = e // _NW
    mesh = plsc.VectorSubcoreMesh(core_axis_name="c", subcore_axis_name="s",
                                  num_cores=_NC, num_subcores=_NS)
    fn = pl.kernel(
        _sc_affine_body,
        out_type=jax.ShapeDtypeStruct((e,), jnp.float32),
        mesh=mesh,
        compiler_params=pltpu.CompilerParams(needs_layout_passes=False),
        scratch_types=[
            pltpu.VMEM((e_per_w * 2,), jnp.int32),
            pltpu.VMEM((e_per_w,), jnp.float32),
            pltpu.VMEM((e_per_w,), jnp.float32),
            pltpu.VMEM((_NUM_PAIR,), jnp.float32),
            pltpu.VMEM((_NUM_PAIR,), jnp.float32),
            pltpu.SemaphoreType.DMA((4,)),
        ],
    )
    return fn(pairs, x_flat, mul_t, bias_t)


_NBUF = 4


def _tc_expand_body(nsteps, y_ref, o_hbm, o_buf, sems):
    i = pl.program_id(0)
    slot = lax.rem(i, _NBUF)
    r = o_buf.shape[1]

    @pl.when(i >= _NBUF)
    def _wait_prev():
        pltpu.make_async_copy(
            o_buf.at[slot],
            o_hbm.at[pl.ds((i - _NBUF) * r, r)],
            sems.at[slot]).wait()

    y = y_ref[...]
    shape = (r, y.shape[1], _K)
    kf = lax.broadcasted_iota(jnp.int32, shape, 2).astype(jnp.float32)
    t = y[:, :, None] * _INV_STD - kf
    o_buf[slot] = jnp.exp(t * t * np.float32(-0.5) + _LN_C)
    pltpu.make_async_copy(
        o_buf.at[slot], o_hbm.at[pl.ds(i * r, r)], sems.at[slot]).start()

    @pl.when(i == nsteps - 1)
    def _drain():
        for s in range(_NBUF):
            step = nsteps - _NBUF + s
            pltpu.make_async_copy(
                o_buf.at[step % _NBUF],
                o_hbm.at[pl.ds(step * r, r)],
                sems.at[step % _NBUF]).wait()


def _tc_expand(y2d, rows_per_block):
    rows, cols = y2d.shape
    nsteps = rows // rows_per_block
    return pl.pallas_call(
        functools.partial(_tc_expand_body, nsteps),
        grid=(nsteps,),
        in_specs=[pl.BlockSpec((rows_per_block, cols), lambda i: (i, 0))],
        out_specs=pl.BlockSpec(memory_space=pl.ANY),
        out_shape=jax.ShapeDtypeStruct((rows, cols, _K), jnp.float32),
        scratch_shapes=[
            pltpu.VMEM((_NBUF, rows_per_block, cols, _K), jnp.float32),
            pltpu.SemaphoreType.DMA((_NBUF,)),
        ],
        compiler_params=pltpu.CompilerParams(
            dimension_semantics=("arbitrary",)),
    )(y2d)


def _sc_full_body(pairs_hbm, x_hbm, mul_hbm, bias_hbm, out_hbm,
                  pairs_v, x_v, y_v, mul_v, bias_v, obuf, sems):
    e_per_w = y_v.shape[0]
    ce = obuf.shape[1] // _K
    nch = e_per_w // ce
    wid = lax.axis_index("s") * _NC + lax.axis_index("c")
    base = wid * e_per_w
    pltpu.sync_copy(pairs_hbm.at[pl.ds(base * 2, e_per_w * 2)], pairs_v)
    pltpu.sync_copy(x_hbm.at[pl.ds(base, e_per_w)], x_v)
    pltpu.sync_copy(mul_hbm, mul_v)
    pltpu.sync_copy(bias_hbm, bias_v)

    lane = lax.iota(jnp.int32, _L)
    lane2 = 2 * lane
    kf = [(lane + 16 * j).astype(jnp.float32) for j in range(_K // _L)]

    def affine(i, _):
        pos0 = i * (2 * _L) + lane2
        i0 = plsc.load_gather(pairs_v, [pos0])
        i1 = plsc.load_gather(pairs_v, [pos0 + 1])
        m = jnp.abs(plsc.load_gather(mul_v, [i0])) + jnp.abs(
            plsc.load_gather(mul_v, [i1]))
        b = plsc.load_gather(bias_v, [i0]) + plsc.load_gather(bias_v, [i1])
        xv = x_v[pl.ds(i * _L, _L)] * _INV_STD
        y_v[pl.ds(i * _L, _L)] = m * xv + b * _INV_STD
        return 0

    lax.fori_loop(0, e_per_w // _L, affine, 0)

    def out_copy(ch, slot):
        return pltpu.make_async_copy(
            obuf.at[slot],
            out_hbm.at[pl.ds((base + ch * ce) * _K, ce * _K)],
            sems.at[slot])

    for ch in range(nch):
        slot = ch % 2

        if ch >= 2:
            out_copy(ch - 2, slot).wait()

        def expand(el, _):
            eidx = jnp.full((_L,), ch * ce + el, jnp.int32)
            yb = plsc.load_gather(y_v, [eidx])
            for j in range(_K // _L):
                t = yb - kf[j]
                obuf[slot, pl.ds(el * _K + j * _L, _L)] = jnp.exp(
                    t * t * np.float32(-0.5) + _LN_C)
            return 0

        lax.fori_loop(0, ce, expand, 0)
        out_copy(ch, slot).start()

    for ch in (nch - 2, nch - 1):
        out_copy(ch, ch % 2).wait()


def _sc_full(pairs, x_flat, mul_t, bias_t, ce=256):
    e = x_flat.shape[0]
    e_per_w = e // _NW
    mesh = plsc.VectorSubcoreMesh(core_axis_name="c", subcore_axis_name="s",
                                  num_cores=_NC, num_subcores=_NS)
    fn = pl.kernel(
        _sc_full_body,
        out_type=jax.ShapeDtypeStruct((e * _K,), jnp.float32),
        mesh=mesh,
        compiler_params=pltpu.CompilerParams(needs_layout_passes=False),
        scratch_types=[
            pltpu.VMEM((e_per_w * 2,), jnp.int32),
            pltpu.VMEM((e_per_w,), jnp.float32),
            pltpu.VMEM((e_per_w,), jnp.float32),
            pltpu.VMEM((_NUM_PAIR,), jnp.float32),
            pltpu.VMEM((_NUM_PAIR,), jnp.float32),
            pltpu.VMEM((2, ce * _K), jnp.float32),
            pltpu.SemaphoreType.DMA((2,)),
        ],
    )
    return fn(pairs, x_flat, mul_t, bias_t)


def kernel(x, atom_pair, mul_weight, bias_weight):
    b, n = x.shape[0], x.shape[1]
    e = b * n * n
    pairs = atom_pair.reshape(e * 2)
    x_flat = x.reshape(e)
    mul_t = mul_weight.reshape(_NUM_PAIR)
    bias_t = bias_weight.reshape(_NUM_PAIR)
    y = _sc_affine(pairs, x_flat, mul_t, bias_t)
    out = _tc_expand(y.reshape(b * n, n), rows_per_block=32)
    return out.reshape(b, n, n, _K)


# speedup vs baseline: 1.8096x; 1.0001x over previous
import functools

import jax
import jax.numpy as jnp
import numpy as np
from jax import lax
from jax.experimental import pallas as pl
from jax.experimental.pallas import tpu as pltpu
from jax.experimental.pallas import tpu_sc as plsc

_K = 128
_NUM_PAIR = 512
_START = 0.0
_STOP = 9.0

_MEAN = np.linspace(_START, _STOP, _K).astype(np.float32)
_STD = np.float32(_MEAN[1] - _MEAN[0])
_INV_STD = np.float32(1.0) / _STD
_A = np.float32((2.0 * 3.14159) ** 0.5)
_LN_C = np.float32(np.log(1.0 / (_A * _STD)))

_NC = 2
_NS = 16
_NW = _NC * _NS
_L = 16

_NBUF = 4


def _sc_affine_body(pairs_hbm, x_hbm, mul_hbm, bias_hbm, y_hbm,
                    pairs_v, x_v, y_v, mul_v, bias_v, insems):
    e_per_w = y_v.shape[0]
    wid = lax.axis_index("s") * _NC + lax.axis_index("c")
    base = wid * e_per_w
    copies = [
        pltpu.async_copy(pairs_hbm.at[pl.ds(base * 2, e_per_w * 2)],
                         pairs_v, insems.at[0]),
        pltpu.async_copy(x_hbm.at[pl.ds(base, e_per_w)], x_v, insems.at[1]),
        pltpu.async_copy(mul_hbm, mul_v, insems.at[2]),
        pltpu.async_copy(bias_hbm, bias_v, insems.at[3]),
    ]
    for c in copies:
        c.wait()

    lane2 = 2 * lax.iota(jnp.int32, _L)

    @plsc.parallel_loop(0, e_per_w // _L, unroll=8)
    def _gather_affine(i):
        pos0 = i * (2 * _L) + lane2
        i0 = plsc.load_gather(pairs_v, [pos0])
        i1 = plsc.load_gather(pairs_v, [pos0 + 1])
        m = jnp.abs(plsc.load_gather(mul_v, [i0])) + jnp.abs(
            plsc.load_gather(mul_v, [i1]))
        b = plsc.load_gather(bias_v, [i0]) + plsc.load_gather(bias_v, [i1])
        xv = x_v[pl.ds(i * _L, _L)]
        y_v[pl.ds(i * _L, _L)] = m * xv + b

    pltpu.sync_copy(y_v, y_hbm.at[pl.ds(base, e_per_w)])


def _sc_affine(pairs, x_flat, mul_t, bias_t):
    e = x_flat.shape[0]
    e_per_w = e // _NW
    mesh = plsc.VectorSubcoreMesh(core_axis_name="c", subcore_axis_name="s",
                                  num_cores=_NC, num_subcores=_NS)
    fn = pl.kernel(
        _sc_affine_body,
        out_type=jax.ShapeDtypeStruct((e,), jnp.float32),
        mesh=mesh,
        compiler_params=pltpu.CompilerParams(needs_layout_passes=False),
        scratch_types=[
            pltpu.VMEM((e_per_w * 2,), jnp.int32),
            pltpu.VMEM((e_per_w,), jnp.float32),
            pltpu.VMEM((e_per_w,), jnp.float32),
            pltpu.VMEM((_NUM_PAIR,), jnp.float32),
            pltpu.VMEM((_NUM_PAIR,), jnp.float32),
            pltpu.SemaphoreType.DMA((4,)),
        ],
    )
    return fn(pairs, x_flat, mul_t, bias_t)


def _tc_expand_body(nsteps, y_ref, o_hbm, o_buf, sems):
    i = pl.program_id(0)
    slot = lax.rem(i, _NBUF)
    r = o_buf.shape[1]

    @pl.when(i >= _NBUF)
    def _wait_prev():
        pltpu.make_async_copy(
            o_buf.at[slot],
            o_hbm.at[pl.ds((i - _NBUF) * r, r)],
            sems.at[slot]).wait()

    y = y_ref[...]
    shape = (r, y.shape[1], _K)
    kf = lax.broadcasted_iota(jnp.int32, shape, 2).astype(jnp.float32)
    t = y[:, :, None] * _INV_STD - kf
    o_buf[slot] = jnp.exp(t * t * np.float32(-0.5) + _LN_C)
    pltpu.make_async_copy(
        o_buf.at[slot], o_hbm.at[pl.ds(i * r, r)], sems.at[slot]).start()

    @pl.when(i == nsteps - 1)
    def _drain():
        for s in range(_NBUF):
            step = nsteps - _NBUF + s
            pltpu.make_async_copy(
                o_buf.at[step % _NBUF],
                o_hbm.at[pl.ds(step * r, r)],
                sems.at[step % _NBUF]).wait()


def _tc_expand(y2d, rows_per_block):
    rows, cols = y2d.shape
    nsteps = rows // rows_per_block
    return pl.pallas_call(
        functools.partial(_tc_expand_body, nsteps),
        grid=(nsteps,),
        in_specs=[pl.BlockSpec((rows_per_block, cols), lambda i: (i, 0))],
        out_specs=pl.BlockSpec(memory_space=pl.ANY),
        out_shape=jax.ShapeDtypeStruct((rows, cols, _K), jnp.float32),
        scratch_shapes=[
            pltpu.VMEM((_NBUF, rows_per_block, cols, _K), jnp.float32),
            pltpu.SemaphoreType.DMA((_NBUF,)),
        ],
        compiler_params=pltpu.CompilerParams(
            dimension_semantics=("arbitrary",)),
    )(y2d)


def kernel(x, atom_pair, mul_weight, bias_weight):
    b, n = x.shape[0], x.shape[1]
    e = b * n * n
    pairs = atom_pair.reshape(e * 2)
    x_flat = x.reshape(e)
    mul_t = mul_weight.reshape(_NUM_PAIR)
    bias_t = bias_weight.reshape(_NUM_PAIR)
    y = _sc_affine(pairs, x_flat, mul_t, bias_t)
    out = _tc_expand(y.reshape(b * n, n), rows_per_block=32)
    return out.reshape(b, n, n, _K)
